# baseline (speedup 1.0000x reference)
import functools

import jax
import jax.numpy as jnp
from jax import lax
from jax.experimental import pallas as pl
from jax.experimental.pallas import tpu as pltpu
from jax.experimental.pallas import tpu_sc as plsc

B, S, D = 4, 4096, 2048
ROWS = B * S
NC, NS = 2, 16
NW = NC * NS
ROWS_W = ROWS // NW
LANES = 16
GROUP = LANES
NGROUP = ROWS_W // GROUP
UNROLL = 4

_mesh = plsc.VectorSubcoreMesh(core_axis_name="c", subcore_axis_name="s")


@functools.partial(
    pl.kernel,
    mesh=_mesh,
    out_type=jax.ShapeDtypeStruct((ROWS, D), jnp.float32),
    scratch_types=[
        pltpu.VMEM((GROUP, D), jnp.float32),
        pltpu.VMEM((GROUP, D), jnp.float32),
        pltpu.SemaphoreType.DMA,
        pltpu.SemaphoreType.DMA,
        pltpu.SemaphoreType.DMA,
        pltpu.SemaphoreType.DMA,
    ],
    compiler_params=pltpu.CompilerParams(needs_layout_passes=False),
)
def _cumsum_rows(x_hbm, out_hbm, buf0, buf1, isem0, isem1, osem0, osem1):
    wid = lax.axis_index("s") * NC + lax.axis_index("c")
    base = wid * ROWS_W
    bufs = (buf0, buf1)
    isems = (isem0, isem1)
    osems = (osem0, osem1)

    def in_copy(g, s):
        return pltpu.make_async_copy(
            x_hbm.at[pl.ds(base + g * GROUP, GROUP)], bufs[s], isems[s])

    def out_copy(g, s):
        return pltpu.make_async_copy(
            bufs[s], out_hbm.at[pl.ds(base + g * GROUP, GROUP)], osems[s])

    riota = lax.iota(jnp.int32, LANES)

    def compute(buf):
        def cbody(jj, acc):
            j = jj * UNROLL
            cols = [jnp.full((LANES,), 0, jnp.int32) + (j + k)
                    for k in range(UNROLL)]
            vs = [plsc.load_gather(buf, [riota, cols[k]])
                  for k in range(UNROLL)]
            p1 = vs[0] + vs[1]
            p2 = p1 + vs[2]
            p3 = p1 + (vs[2] + vs[3])
            outs = [acc + vs[0], acc + p1, acc + p2, acc + p3]
            for k in range(UNROLL):
                plsc.store_scatter(buf, [riota, cols[k]], outs[k])
            return outs[UNROLL - 1]

        lax.fori_loop(0, D // UNROLL, cbody, jnp.zeros((LANES,), jnp.float32))

    in_copy(0, 0).start()
    in_copy(1, 1).start()

    def gbody(gg, carry):
        for s in range(2):
            g = gg * 2 + s

            in_copy(g, s).wait()

            @pl.when(gg > 0)
            def _():
                out_copy(g - 2, s).wait()

            compute(bufs[s])
            out_copy(g, s).start()

            @pl.when(g + 2 < NGROUP)
            def _():
                in_copy(g + 2, s).start()
        return carry

    lax.fori_loop(0, NGROUP // 2, gbody, 0)
    out_copy(NGROUP - 2, 0).wait()
    out_copy(NGROUP - 1, 1).wait()


def kernel(x):
    out = _cumsum_rows(x.reshape(ROWS, D))
    return out.reshape(B, S, D)


# speedup vs baseline: 1.0002x; 1.0002x over previous
import functools

import jax
import jax.numpy as jnp
from jax import lax
from jax.experimental import pallas as pl
from jax.experimental.pallas import tpu as pltpu
from jax.experimental.pallas import tpu_sc as plsc

B, S, D = 4, 4096, 2048
ROWS = B * S
NC, NS = 2, 16
NW = NC * NS
ROWS_W = ROWS // NW
LANES = 16
GROUP = LANES
NGROUP = ROWS_W // GROUP
UNROLL = 4
DPAD = D + 1

_mesh = plsc.VectorSubcoreMesh(core_axis_name="c", subcore_axis_name="s")


@functools.partial(
    pl.kernel,
    mesh=_mesh,
    out_type=jax.ShapeDtypeStruct((ROWS, D), jnp.float32),
    scratch_types=[
        pltpu.VMEM((GROUP, DPAD), jnp.float32),
        pltpu.VMEM((GROUP, DPAD), jnp.float32),
        pltpu.SemaphoreType.DMA,
        pltpu.SemaphoreType.DMA,
        pltpu.SemaphoreType.DMA,
        pltpu.SemaphoreType.DMA,
    ],
    compiler_params=pltpu.CompilerParams(needs_layout_passes=False),
)
def _cumsum_rows(x_hbm, out_hbm, buf0, buf1, isem0, isem1, osem0, osem1):
    wid = lax.axis_index("s") * NC + lax.axis_index("c")
    base = wid * ROWS_W
    bufs = (buf0, buf1)
    isems = (isem0, isem1)
    osems = (osem0, osem1)

    def in_copy(g, s):
        return pltpu.make_async_copy(
            x_hbm.at[pl.ds(base + g * GROUP, GROUP)],
            bufs[s].at[:, pl.ds(0, D)], isems[s])

    def out_copy(g, s):
        return pltpu.make_async_copy(
            bufs[s].at[:, pl.ds(0, D)],
            out_hbm.at[pl.ds(base + g * GROUP, GROUP)], osems[s])

    riota = lax.iota(jnp.int32, LANES)

    def compute(buf):
        def cbody(jj, acc):
            j = jj * UNROLL
            cols = [jnp.full((LANES,), 0, jnp.int32) + (j + k)
                    for k in range(UNROLL)]
            vs = [plsc.load_gather(buf, [riota, cols[k]])
                  for k in range(UNROLL)]
            p1 = vs[0] + vs[1]
            p2 = p1 + vs[2]
            p3 = p1 + (vs[2] + vs[3])
            outs = [acc + vs[0], acc + p1, acc + p2, acc + p3]
            for k in range(UNROLL):
                plsc.store_scatter(buf, [riota, cols[k]], outs[k])
            return outs[UNROLL - 1]

        lax.fori_loop(0, D // UNROLL, cbody, jnp.zeros((LANES,), jnp.float32))

    in_copy(0, 0).start()
    in_copy(1, 1).start()

    def gbody(gg, carry):
        for s in range(2):
            g = gg * 2 + s

            in_copy(g, s).wait()

            @pl.when(gg > 0)
            def _():
                out_copy(g - 2, s).wait()

            compute(bufs[s])
            out_copy(g, s).start()

            @pl.when(g + 2 < NGROUP)
            def _():
                in_copy(g + 2, s).start()
        return carry

    lax.fori_loop(0, NGROUP // 2, gbody, 0)
    out_copy(NGROUP - 2, 0).wait()
    out_copy(NGROUP - 1, 1).wait()


def kernel(x):
    out = _cumsum_rows(x.reshape(ROWS, D))
    return out.reshape(B, S, D)


# speedup vs baseline: 3.4475x; 3.4468x over previous
import functools

import jax
import jax.numpy as jnp
from jax import lax
from jax.experimental import pallas as pl
from jax.experimental.pallas import tpu as pltpu
from jax.experimental.pallas import tpu_sc as plsc

B, S, D = 4, 4096, 2048
ROWS = B * S
NC, NS = 2, 16
NW = NC * NS
ROWS_W = ROWS // NW
LANES = 16
GROUP = 8
NGROUP = ROWS_W // GROUP
NV = D // LANES

_mesh = plsc.VectorSubcoreMesh(core_axis_name="c", subcore_axis_name="s")


@functools.partial(
    pl.kernel,
    mesh=_mesh,
    out_type=jax.ShapeDtypeStruct((ROWS, D), jnp.float32),
    scratch_types=[
        pltpu.VMEM((GROUP, D), jnp.float32),
        pltpu.VMEM((GROUP, D), jnp.float32),
        pltpu.SemaphoreType.DMA,
        pltpu.SemaphoreType.DMA,
        pltpu.SemaphoreType.DMA,
        pltpu.SemaphoreType.DMA,
    ],
    compiler_params=pltpu.CompilerParams(needs_layout_passes=False),
)
def _cumsum_rows(x_hbm, out_hbm, buf0, buf1, isem0, isem1, osem0, osem1):
    wid = lax.axis_index("s") * NC + lax.axis_index("c")
    base = wid * ROWS_W
    bufs = (buf0, buf1)
    isems = (isem0, isem1)
    osems = (osem0, osem1)

    def in_copy(g, s):
        return pltpu.make_async_copy(
            x_hbm.at[pl.ds(base + g * GROUP, GROUP)], bufs[s], isems[s])

    def out_copy(g, s):
        return pltpu.make_async_copy(
            bufs[s], out_hbm.at[pl.ds(base + g * GROUP, GROUP)], osems[s])

    def compute(buf):
        def step(i, carries):
            off = i * LANES
            new = []
            for r in range(GROUP):
                v = buf[r, pl.ds(off, LANES)]
                s = plsc.cumsum(v)
                suf = lax.rev(plsc.cumsum(lax.rev(v, (0,))), (0,))
                total = (s - v) + suf
                buf[r, pl.ds(off, LANES)] = s + carries[r]
                new.append(carries[r] + total)
            return tuple(new)

        lax.fori_loop(
            0, NV, step,
            tuple(jnp.zeros((LANES,), jnp.float32) for _ in range(GROUP)))

    in_copy(0, 0).start()
    in_copy(1, 1).start()

    def gbody(gg, carry):
        for s in range(2):
            g = gg * 2 + s

            in_copy(g, s).wait()

            @pl.when(gg > 0)
            def _():
                out_copy(g - 2, s).wait()

            compute(bufs[s])
            out_copy(g, s).start()

            @pl.when(g + 2 < NGROUP)
            def _():
                in_copy(g + 2, s).start()
        return carry

    lax.fori_loop(0, NGROUP // 2, gbody, 0)
    out_copy(NGROUP - 2, 0).wait()
    out_copy(NGROUP - 1, 1).wait()


def kernel(x):
    out = _cumsum_rows(x.reshape(ROWS, D))
    return out.reshape(B, S, D)


# speedup vs baseline: 3.4995x; 1.0151x over previous
import functools

import jax
import jax.numpy as jnp
from jax import lax
from jax.experimental import pallas as pl
from jax.experimental.pallas import tpu as pltpu
from jax.experimental.pallas import tpu_sc as plsc

B, S, D = 4, 4096, 2048
ROWS = B * S
NC, NS = 2, 16
NW = NC * NS
ROWS_W = ROWS // NW
LANES = 16
GROUP = 8
NGROUP = ROWS_W // GROUP
NV = D // LANES

_mesh = plsc.VectorSubcoreMesh(core_axis_name="c", subcore_axis_name="s")


@functools.partial(
    pl.kernel,
    mesh=_mesh,
    out_type=jax.ShapeDtypeStruct((ROWS, D), jnp.float32),
    scratch_types=[
        pltpu.VMEM((GROUP, D), jnp.float32),
        pltpu.VMEM((GROUP, D), jnp.float32),
        pltpu.SemaphoreType.DMA,
        pltpu.SemaphoreType.DMA,
        pltpu.SemaphoreType.DMA,
        pltpu.SemaphoreType.DMA,
    ],
    compiler_params=pltpu.CompilerParams(needs_layout_passes=False),
)
def _cumsum_rows(x_hbm, out_hbm, buf0, buf1, isem0, isem1, osem0, osem1):
    wid = lax.axis_index("s") * NC + lax.axis_index("c")
    base = wid * ROWS_W
    bufs = (buf0, buf1)
    isems = (isem0, isem1)
    osems = (osem0, osem1)

    def in_copy(g, s):
        return pltpu.make_async_copy(
            x_hbm.at[pl.ds(base + g * GROUP, GROUP)], bufs[s], isems[s])

    def out_copy(g, s):
        return pltpu.make_async_copy(
            bufs[s], out_hbm.at[pl.ds(base + g * GROUP, GROUP)], osems[s])

    lane0 = lax.iota(jnp.int32, LANES) == 0

    def compute(buf):
        def step(i, carries):
            off = i * LANES
            new = []
            for r in range(GROUP):
                v = buf[r, pl.ds(off, LANES)]
                s = plsc.cumsum(v)
                total = plsc.cumsum(lax.rev(s, (0,)), mask=lane0)
                buf[r, pl.ds(off, LANES)] = s + carries[r]
                new.append(carries[r] + total)
            return tuple(new)

        lax.fori_loop(
            0, NV, step,
            tuple(jnp.zeros((LANES,), jnp.float32) for _ in range(GROUP)))

    in_copy(0, 0).start()
    in_copy(1, 1).start()

    def gbody(gg, carry):
        for s in range(2):
            g = gg * 2 + s

            in_copy(g, s).wait()

            @pl.when(gg > 0)
            def _():
                out_copy(g - 2, s).wait()

            compute(bufs[s])
            out_copy(g, s).start()

            @pl.when(g + 2 < NGROUP)
            def _():
                in_copy(g + 2, s).start()
        return carry

    lax.fori_loop(0, NGROUP // 2, gbody, 0)
    out_copy(NGROUP - 2, 0).wait()
    out_copy(NGROUP - 1, 1).wait()


def kernel(x):
    out = _cumsum_rows(x.reshape(ROWS, D))
    return out.reshape(B, S, D)


# speedup vs baseline: 3.6261x; 1.0362x over previous
import functools

import jax
import jax.numpy as jnp
from jax import lax
from jax.experimental import pallas as pl
from jax.experimental.pallas import tpu as pltpu
from jax.experimental.pallas import tpu_sc as plsc

B, S, D = 4, 4096, 2048
ROWS = B * S
NC, NS = 2, 16
NW = NC * NS
ROWS_W = ROWS // NW
LANES = 16
GROUP = 16
NGROUP = ROWS_W // GROUP
NV = D // LANES

_mesh = plsc.VectorSubcoreMesh(core_axis_name="c", subcore_axis_name="s")


@functools.partial(
    pl.kernel,
    mesh=_mesh,
    out_type=jax.ShapeDtypeStruct((ROWS, D), jnp.float32),
    scratch_types=[
        pltpu.VMEM((GROUP, D), jnp.float32),
        pltpu.VMEM((GROUP, D), jnp.float32),
        pltpu.SemaphoreType.DMA,
        pltpu.SemaphoreType.DMA,
        pltpu.SemaphoreType.DMA,
        pltpu.SemaphoreType.DMA,
    ],
    compiler_params=pltpu.CompilerParams(needs_layout_passes=False),
)
def _cumsum_rows(x_hbm, out_hbm, buf0, buf1, isem0, isem1, osem0, osem1):
    wid = lax.axis_index("s") * NC + lax.axis_index("c")
    base = wid * ROWS_W
    bufs = (buf0, buf1)
    isems = (isem0, isem1)
    osems = (osem0, osem1)

    def in_copy(g, s):
        return pltpu.make_async_copy(
            x_hbm.at[pl.ds(base + g * GROUP, GROUP)], bufs[s], isems[s])

    def out_copy(g, s):
        return pltpu.make_async_copy(
            bufs[s], out_hbm.at[pl.ds(base + g * GROUP, GROUP)], osems[s])

    lane0 = lax.iota(jnp.int32, LANES) == 0

    def compute(buf):
        def step(i, carries):
            off = i * LANES
            new = []
            for r in range(GROUP):
                v = buf[r, pl.ds(off, LANES)]
                s = plsc.cumsum(v)
                total = plsc.cumsum(lax.rev(s, (0,)), mask=lane0)
                buf[r, pl.ds(off, LANES)] = s + carries[r]
                new.append(carries[r] + total)
            return tuple(new)

        lax.fori_loop(
            0, NV, step,
            tuple(jnp.zeros((LANES,), jnp.float32) for _ in range(GROUP)))

    in_copy(0, 0).start()
    in_copy(1, 1).start()

    def gbody(gg, carry):
        for s in range(2):
            g = gg * 2 + s

            in_copy(g, s).wait()

            @pl.when(gg > 0)
            def _():
                out_copy(g - 2, s).wait()

            compute(bufs[s])
            out_copy(g, s).start()

            @pl.when(g + 2 < NGROUP)
            def _():
                in_copy(g + 2, s).start()
        return carry

    lax.fori_loop(0, NGROUP // 2, gbody, 0)
    out_copy(NGROUP - 2, 0).wait()
    out_copy(NGROUP - 1, 1).wait()


def kernel(x):
    out = _cumsum_rows(x.reshape(ROWS, D))
    return out.reshape(B, S, D)
